# baseline (speedup 1.0000x reference)
import functools

import jax
import jax.numpy as jnp
from jax import lax
from jax.experimental import pallas as pl
from jax.experimental.pallas import tpu as pltpu
from jax.experimental.pallas import tpu_sc as plsc

N = 10000
E = 320000
D = 128
DH = 64
DO = 64

NC = 2
NS = 16

CH = 125
EPT = E // NS
NCHUNK = EPT // CH
NBUF = 4

DNCHUNK = NCHUNK // NC

NP = 10240
SLAB = NP // NS

BLK = 1024
NBLK = NP // BLK


NR = NP // 128
RPT = NR // NS


def _sc_degree_body(ei_hbm, zeros_hbm, out_hbm, dst_v, cnt_v, rix_v, deg_sh):
    c = lax.axis_index("c")
    s = lax.axis_index("s")
    f32 = jnp.float32
    zeros16 = jnp.zeros((16,), f32)
    ones16 = jnp.ones((16,), f32)
    iota16 = lax.broadcasted_iota(jnp.int32, (16,), 0)

    def zrow(i, carry):
        for k in range(8):
            cnt_v[i, pl.ds(k * 16, 16)] = zeros16
        return carry

    lax.fori_loop(0, NR, zrow, 0)
    for k in range(NR // 16):
        rix_v[pl.ds(k * 16, 16)] = iota16 + (k * 16)
    pltpu.sync_copy(zeros_hbm.at[pl.ds(s * RPT, RPT)], deg_sh.at[pl.ds(s * RPT, RPT)])
    pltpu.sync_copy(ei_hbm.at[1, s, pl.ds(c * DNCHUNK, DNCHUNK)], dst_v)
    tailmask = iota16 >= 3

    def body(j, carry):
        for k in range(7):
            idx = dst_v[j, pl.ds(k * 16, 16)]
            plsc.addupdate_scatter(cnt_v, [idx >> 7, idx & 127], ones16)
        idx = dst_v[j, pl.ds(CH - 16, 16)]
        plsc.addupdate_scatter(cnt_v, [idx >> 7, idx & 127], ones16,
                               mask=tailmask)
        return carry

    lax.fori_loop(0, DNCHUNK, body, 0)
    plsc.subcore_barrier()
    pltpu.sync_copy(cnt_v, deg_sh.at[rix_v], add=True)
    plsc.subcore_barrier()
    pltpu.sync_copy(deg_sh.at[pl.ds(s * RPT, RPT)], out_hbm.at[c, pl.ds(s * RPT, RPT)])


def _sc_edge_body(g_hbm, ei_hbm, zeros_hbm, out_hbm,
                  src_v, dst_v, rows, gsems, ssems, acc_sh):
    c = lax.axis_index("c")
    s = lax.axis_index("s")
    slab = s * SLAB
    pltpu.sync_copy(zeros_hbm.at[pl.ds(slab, SLAB)], acc_sh.at[pl.ds(slab, SLAB)])
    pltpu.sync_copy(ei_hbm.at[0, s], src_v)
    pltpu.sync_copy(ei_hbm.at[1, s], dst_v)
    plsc.subcore_barrier()

    table = g_hbm.at[c]

    for b in range(NBUF):
        pltpu.async_copy(table.at[src_v.at[b]], rows[b], gsems[b])

    def body(grp, carry):
        for b in range(NBUF):
            ch = grp * NBUF + b

            @pl.when(grp >= 1)
            def _():
                pltpu.make_async_copy(rows[b], acc_sh.at[dst_v.at[0]],
                                      ssems[b]).wait()
                pltpu.async_copy(table.at[src_v.at[ch]], rows[b], gsems[b])

            pltpu.make_async_copy(table.at[src_v.at[ch]], rows[b],
                                  gsems[b]).wait()
            pltpu.async_copy(rows[b], acc_sh.at[dst_v.at[ch]], ssems[b],
                             add=True)
        return carry

    lax.fori_loop(0, NCHUNK // NBUF, body, 0)
    for b in range(NBUF):
        pltpu.make_async_copy(rows[b], acc_sh.at[dst_v.at[0]], ssems[b]).wait()
    plsc.subcore_barrier()
    pltpu.sync_copy(acc_sh.at[pl.ds(slab, SLAB)],
                    out_hbm.at[pl.ds(slab, SLAB), pl.ds(c * DH, DH)])


@functools.lru_cache(maxsize=None)
def _sc_kernels():
    mesh = plsc.VectorSubcoreMesh(core_axis_name="c", subcore_axis_name="s",
                                  num_cores=NC, num_subcores=NS)
    params = pltpu.CompilerParams(use_tc_tiling_on_sc=False)
    sc_degree = pl.kernel(
        _sc_degree_body,
        out_type=jax.ShapeDtypeStruct((NC, NR, 128), jnp.float32),
        mesh=mesh,
        compiler_params=pltpu.CompilerParams(use_tc_tiling_on_sc=False,
                                             needs_layout_passes=False),
        scratch_types=[
            pltpu.VMEM((DNCHUNK, CH), jnp.int32),
            pltpu.VMEM((NR, 128), jnp.float32),
            pltpu.VMEM((NR,), jnp.int32),
            pltpu.VMEM_SHARED((NR, 128), jnp.float32),
        ],
    )
    sc_edge = pl.kernel(
        _sc_edge_body,
        out_type=jax.ShapeDtypeStruct((NP, D), jnp.bfloat16),
        mesh=mesh,
        compiler_params=params,
        scratch_types=[
            pltpu.VMEM((NCHUNK, CH), jnp.int32),
            pltpu.VMEM((NCHUNK, CH), jnp.int32),
            [pltpu.VMEM((CH, DH), jnp.bfloat16) for _ in range(NBUF)],
            [pltpu.SemaphoreType.DMA for _ in range(NBUF)],
            [pltpu.SemaphoreType.DMA for _ in range(NBUF)],
            pltpu.VMEM_SHARED((NP, DH), jnp.bfloat16),
        ],
    )
    return sc_degree, sc_edge


def _tc1a_body(x_ref, w1_ref, h_ref):
    h_ref[...] = lax.dot_general(x_ref[...], w1_ref[...],
                                 (((1,), (1,)), ((), ())),
                                 preferred_element_type=jnp.float32)


_tc1a = pl.pallas_call(
    _tc1a_body,
    grid=(NBLK,),
    in_specs=[
        pl.BlockSpec((BLK, D), lambda i: (i, 0)),
        pl.BlockSpec((D, D), lambda i: (0, 0)),
    ],
    out_specs=pl.BlockSpec((BLK, D), lambda i: (i, 0)),
    out_shape=jax.ShapeDtypeStruct((NP, D), jnp.float32),
)


def _tc1b_body(deg_ref, h_ref, g_ref, dinv_ref):
    deg = deg_ref[0] + deg_ref[1] + 1.0
    dv = lax.rsqrt(deg)
    g = (h_ref[...] * dv[:, None]).astype(jnp.bfloat16)
    g_ref[0] = g[:, :DH]
    g_ref[1] = g[:, DH:]
    dinv_ref[...] = dv


_tc1b = pl.pallas_call(
    _tc1b_body,
    grid=(NBLK,),
    in_specs=[
        pl.BlockSpec((NC, BLK), lambda i: (0, i)),
        pl.BlockSpec((BLK, D), lambda i: (i, 0)),
    ],
    out_specs=[
        pl.BlockSpec((NC, BLK, DH), lambda i: (0, i, 0)),
        pl.BlockSpec((BLK,), lambda i: (i,)),
    ],
    out_shape=[
        jax.ShapeDtypeStruct((NC, NP, DH), jnp.bfloat16),
        jax.ShapeDtypeStruct((NP,), jnp.float32),
    ],
)


def _tc2_body(t_ref, g_ref, dinv_ref, b1_ref, w2_ref, g2_ref):
    f32 = jnp.float32
    dv = dinv_ref[...]
    gfull = jnp.concatenate([g_ref[0], g_ref[1]], axis=1).astype(f32)
    t = t_ref[...].astype(f32) + gfull
    o1 = jnp.maximum(t * dv[:, None] + b1_ref[...][None, :], 0.0)
    h2 = lax.dot_general(o1, w2_ref[...], (((1,), (1,)), ((), ())),
                         preferred_element_type=jnp.float32)
    g2 = (h2 * dv[:, None]).astype(jnp.bfloat16)
    g2_ref[0] = g2[:, :DH]
    g2_ref[1] = g2[:, DH:]


_tc2 = pl.pallas_call(
    _tc2_body,
    grid=(NBLK,),
    in_specs=[
        pl.BlockSpec((BLK, D), lambda i: (i, 0)),
        pl.BlockSpec((NC, BLK, DH), lambda i: (0, i, 0)),
        pl.BlockSpec((BLK,), lambda i: (i,)),
        pl.BlockSpec((D,), lambda i: (0,)),
        pl.BlockSpec((D, D), lambda i: (0, 0)),
    ],
    out_specs=pl.BlockSpec((NC, BLK, DH), lambda i: (0, i, 0)),
    out_shape=jax.ShapeDtypeStruct((NC, NP, DH), jnp.bfloat16),
)


def _tc3_body(t_ref, g_ref, dinv_ref, b2_ref, wl_ref, bl_ref, out_ref):
    i = pl.program_id(0)
    f32 = jnp.float32
    dv = dinv_ref[...]
    gfull = jnp.concatenate([g_ref[0], g_ref[1]], axis=1).astype(f32)
    t = t_ref[...].astype(f32) + gfull
    o2 = jnp.maximum(t * dv[:, None] + b2_ref[...][None, :], 0.0)
    y = lax.dot_general(o2, wl_ref[...], (((1,), (1,)), ((), ())),
                        preferred_element_type=jnp.float32)
    y = jnp.maximum(y + bl_ref[...][None, :], 0.0)
    rid = i * BLK + lax.broadcasted_iota(jnp.int32, (BLK, 1), 0)
    y = jnp.where(rid < N, y, 0.0)
    ssum = jnp.sum(y, axis=0, keepdims=True)

    @pl.when(i == 0)
    def _():
        out_ref[...] = ssum

    @pl.when(i > 0)
    def _():
        out_ref[...] = out_ref[...] + ssum

    @pl.when(i == NBLK - 1)
    def _():
        out_ref[...] = jax.nn.sigmoid(out_ref[...])


_tc3 = pl.pallas_call(
    _tc3_body,
    grid=(NBLK,),
    in_specs=[
        pl.BlockSpec((BLK, D), lambda i: (i, 0)),
        pl.BlockSpec((NC, BLK, DH), lambda i: (0, i, 0)),
        pl.BlockSpec((BLK,), lambda i: (i,)),
        pl.BlockSpec((D,), lambda i: (0,)),
        pl.BlockSpec((DO, D), lambda i: (0, 0)),
        pl.BlockSpec((DO,), lambda i: (0,)),
    ],
    out_specs=pl.BlockSpec((1, DO), lambda i: (0, 0)),
    out_shape=jax.ShapeDtypeStruct((1, DO), jnp.float32),
)


def kernel(x, edge_index, batch, W1, b1, W2, b2, Wl, bl):
    f32 = jnp.float32
    eir = edge_index.reshape(2, NS, NCHUNK, CH)
    xp = jnp.concatenate([x.astype(f32), jnp.zeros((NP - N, D), f32)], axis=0)
    zeros_h = jnp.zeros((NP, DH), jnp.bfloat16)
    zeros_r = jnp.zeros((NR, 128), f32)

    sc_degree, sc_edge = _sc_kernels()
    degp = sc_degree(eir, zeros_r).reshape(NC, NP)
    h1 = _tc1a(xp, W1)
    g1, dinv = _tc1b(degp, h1)
    t1 = sc_edge(g1, eir, zeros_h)
    g2 = _tc2(t1, g1, dinv, b1, W2)
    t2 = sc_edge(g2, eir, zeros_h)
    out = _tc3(t2, g2, dinv, b2, Wl, bl)
    return out[0]


# speedup vs baseline: 1.6557x; 1.6557x over previous
import functools

import jax
import jax.numpy as jnp
from jax import lax
from jax.experimental import pallas as pl
from jax.experimental.pallas import tpu as pltpu
from jax.experimental.pallas import tpu_sc as plsc

N = 10000
E = 320000
D = 128
DH = 64
DO = 64

NC = 2
NS = 16

CH = 125
EPT = E // NS
NCHUNK = EPT // CH
NBUF = 4

DNCHUNK = NCHUNK // NC

NP = 10240
SLAB = NP // NS

BLK = 1024
NBLK = NP // BLK


NR = NP // 128
RPT = NR // NS


def _sc_degree_body(ei_hbm, zeros_hbm, out_hbm, dst_v, cnt_v, rix_v, deg_sh):
    c = lax.axis_index("c")
    s = lax.axis_index("s")
    f32 = jnp.float32
    zeros16 = jnp.zeros((16,), f32)
    ones16 = jnp.ones((16,), f32)
    iota16 = lax.broadcasted_iota(jnp.int32, (16,), 0)

    def zrow(i, carry):
        for k in range(8):
            cnt_v[i, pl.ds(k * 16, 16)] = zeros16
        return carry

    lax.fori_loop(0, NR, zrow, 0)
    for k in range(NR // 16):
        rix_v[pl.ds(k * 16, 16)] = iota16 + (k * 16)
    pltpu.sync_copy(zeros_hbm.at[pl.ds(s * RPT, RPT)], deg_sh.at[pl.ds(s * RPT, RPT)])
    pltpu.sync_copy(ei_hbm.at[1, s, pl.ds(c * DNCHUNK, DNCHUNK)], dst_v)
    tailmask = iota16 >= 3

    def body(j, carry):
        for k in range(7):
            idx = dst_v[j, pl.ds(k * 16, 16)]
            plsc.addupdate_scatter(cnt_v, [idx >> 7, idx & 127], ones16)
        idx = dst_v[j, pl.ds(CH - 16, 16)]
        plsc.addupdate_scatter(cnt_v, [idx >> 7, idx & 127], ones16,
                               mask=tailmask)
        return carry

    lax.fori_loop(0, DNCHUNK, body, 0)
    plsc.subcore_barrier()
    pltpu.sync_copy(cnt_v, deg_sh.at[rix_v], add=True)
    plsc.subcore_barrier()
    pltpu.sync_copy(deg_sh.at[pl.ds(s * RPT, RPT)], out_hbm.at[c, pl.ds(s * RPT, RPT)])


def _sc_edge_body(g_hbm, ei_hbm, zeros_hbm, out_hbm,
                  src_v, dst_v, rows, gsems, acc_sh):
    c = lax.axis_index("c")
    s = lax.axis_index("s")
    slab = s * SLAB
    pltpu.sync_copy(zeros_hbm.at[pl.ds(slab, SLAB)], acc_sh.at[pl.ds(slab, SLAB)])
    pltpu.sync_copy(ei_hbm.at[0, s], src_v)
    pltpu.sync_copy(ei_hbm.at[1, s], dst_v)
    plsc.subcore_barrier()

    table = g_hbm.at[c]

    for b in range(NBUF):
        pltpu.async_copy(table.at[src_v.at[b]], rows[b], gsems[b])

    def body(grp, carry):
        for b in range(NBUF):
            ch = grp * NBUF + b
            pltpu.make_async_copy(table.at[src_v.at[ch]], rows[b],
                                  gsems[b]).wait()
            pltpu.sync_copy(rows[b], acc_sh.at[dst_v.at[ch]], add=True)

            @pl.when(ch + NBUF < NCHUNK)
            def _():
                pltpu.async_copy(table.at[src_v.at[ch + NBUF]], rows[b],
                                 gsems[b])
        return carry

    lax.fori_loop(0, NCHUNK // NBUF, body, 0)
    plsc.subcore_barrier()
    pltpu.sync_copy(acc_sh.at[pl.ds(slab, SLAB)],
                    out_hbm.at[pl.ds(slab, SLAB), pl.ds(c * DH, DH)])


@functools.lru_cache(maxsize=None)
def _sc_kernels():
    mesh = plsc.VectorSubcoreMesh(core_axis_name="c", subcore_axis_name="s",
                                  num_cores=NC, num_subcores=NS)
    params = pltpu.CompilerParams(use_tc_tiling_on_sc=False)
    sc_degree = pl.kernel(
        _sc_degree_body,
        out_type=jax.ShapeDtypeStruct((NC, NR, 128), jnp.float32),
        mesh=mesh,
        compiler_params=pltpu.CompilerParams(use_tc_tiling_on_sc=False,
                                             needs_layout_passes=False),
        scratch_types=[
            pltpu.VMEM((DNCHUNK, CH), jnp.int32),
            pltpu.VMEM((NR, 128), jnp.float32),
            pltpu.VMEM((NR,), jnp.int32),
            pltpu.VMEM_SHARED((NR, 128), jnp.float32),
        ],
    )
    sc_edge = pl.kernel(
        _sc_edge_body,
        out_type=jax.ShapeDtypeStruct((NP, D), jnp.bfloat16),
        mesh=mesh,
        compiler_params=params,
        scratch_types=[
            pltpu.VMEM((NCHUNK, CH), jnp.int32),
            pltpu.VMEM((NCHUNK, CH), jnp.int32),
            [pltpu.VMEM((CH, DH), jnp.bfloat16) for _ in range(NBUF)],
            [pltpu.SemaphoreType.DMA for _ in range(NBUF)],
            pltpu.VMEM_SHARED((NP, DH), jnp.bfloat16),
        ],
    )
    return sc_degree, sc_edge


def _tc1a_body(x_ref, w1_ref, h_ref):
    h_ref[...] = lax.dot_general(x_ref[...], w1_ref[...],
                                 (((1,), (1,)), ((), ())),
                                 preferred_element_type=jnp.float32)


_tc1a = pl.pallas_call(
    _tc1a_body,
    grid=(NBLK,),
    in_specs=[
        pl.BlockSpec((BLK, D), lambda i: (i, 0)),
        pl.BlockSpec((D, D), lambda i: (0, 0)),
    ],
    out_specs=pl.BlockSpec((BLK, D), lambda i: (i, 0)),
    out_shape=jax.ShapeDtypeStruct((NP, D), jnp.float32),
)


def _tc1b_body(deg_ref, h_ref, g_ref, dinv_ref):
    deg = deg_ref[0] + deg_ref[1] + 1.0
    dv = lax.rsqrt(deg)
    g = (h_ref[...] * dv[:, None]).astype(jnp.bfloat16)
    g_ref[0] = g[:, :DH]
    g_ref[1] = g[:, DH:]
    dinv_ref[...] = dv


_tc1b = pl.pallas_call(
    _tc1b_body,
    grid=(NBLK,),
    in_specs=[
        pl.BlockSpec((NC, BLK), lambda i: (0, i)),
        pl.BlockSpec((BLK, D), lambda i: (i, 0)),
    ],
    out_specs=[
        pl.BlockSpec((NC, BLK, DH), lambda i: (0, i, 0)),
        pl.BlockSpec((BLK,), lambda i: (i,)),
    ],
    out_shape=[
        jax.ShapeDtypeStruct((NC, NP, DH), jnp.bfloat16),
        jax.ShapeDtypeStruct((NP,), jnp.float32),
    ],
)


def _tc2_body(t_ref, g_ref, dinv_ref, b1_ref, w2_ref, g2_ref):
    f32 = jnp.float32
    dv = dinv_ref[...]
    gfull = jnp.concatenate([g_ref[0], g_ref[1]], axis=1).astype(f32)
    t = t_ref[...].astype(f32) + gfull
    o1 = jnp.maximum(t * dv[:, None] + b1_ref[...][None, :], 0.0)
    h2 = lax.dot_general(o1, w2_ref[...], (((1,), (1,)), ((), ())),
                         preferred_element_type=jnp.float32)
    g2 = (h2 * dv[:, None]).astype(jnp.bfloat16)
    g2_ref[0] = g2[:, :DH]
    g2_ref[1] = g2[:, DH:]


_tc2 = pl.pallas_call(
    _tc2_body,
    grid=(NBLK,),
    in_specs=[
        pl.BlockSpec((BLK, D), lambda i: (i, 0)),
        pl.BlockSpec((NC, BLK, DH), lambda i: (0, i, 0)),
        pl.BlockSpec((BLK,), lambda i: (i,)),
        pl.BlockSpec((D,), lambda i: (0,)),
        pl.BlockSpec((D, D), lambda i: (0, 0)),
    ],
    out_specs=pl.BlockSpec((NC, BLK, DH), lambda i: (0, i, 0)),
    out_shape=jax.ShapeDtypeStruct((NC, NP, DH), jnp.bfloat16),
)


def _tc3_body(t_ref, g_ref, dinv_ref, b2_ref, wl_ref, bl_ref, out_ref):
    i = pl.program_id(0)
    f32 = jnp.float32
    dv = dinv_ref[...]
    gfull = jnp.concatenate([g_ref[0], g_ref[1]], axis=1).astype(f32)
    t = t_ref[...].astype(f32) + gfull
    o2 = jnp.maximum(t * dv[:, None] + b2_ref[...][None, :], 0.0)
    y = lax.dot_general(o2, wl_ref[...], (((1,), (1,)), ((), ())),
                        preferred_element_type=jnp.float32)
    y = jnp.maximum(y + bl_ref[...][None, :], 0.0)
    rid = i * BLK + lax.broadcasted_iota(jnp.int32, (BLK, 1), 0)
    y = jnp.where(rid < N, y, 0.0)
    ssum = jnp.sum(y, axis=0, keepdims=True)

    @pl.when(i == 0)
    def _():
        out_ref[...] = ssum

    @pl.when(i > 0)
    def _():
        out_ref[...] = out_ref[...] + ssum

    @pl.when(i == NBLK - 1)
    def _():
        out_ref[...] = jax.nn.sigmoid(out_ref[...])


_tc3 = pl.pallas_call(
    _tc3_body,
    grid=(NBLK,),
    in_specs=[
        pl.BlockSpec((BLK, D), lambda i: (i, 0)),
        pl.BlockSpec((NC, BLK, DH), lambda i: (0, i, 0)),
        pl.BlockSpec((BLK,), lambda i: (i,)),
        pl.BlockSpec((D,), lambda i: (0,)),
        pl.BlockSpec((DO, D), lambda i: (0, 0)),
        pl.BlockSpec((DO,), lambda i: (0,)),
    ],
    out_specs=pl.BlockSpec((1, DO), lambda i: (0, 0)),
    out_shape=jax.ShapeDtypeStruct((1, DO), jnp.float32),
)


def kernel(x, edge_index, batch, W1, b1, W2, b2, Wl, bl):
    f32 = jnp.float32
    eir = edge_index.reshape(2, NS, NCHUNK, CH)
    xp = jnp.concatenate([x.astype(f32), jnp.zeros((NP - N, D), f32)], axis=0)
    zeros_h = jnp.zeros((NP, DH), jnp.bfloat16)
    zeros_r = jnp.zeros((NR, 128), f32)

    sc_degree, sc_edge = _sc_kernels()
    degp = sc_degree(eir, zeros_r).reshape(NC, NP)
    h1 = _tc1a(xp, W1)
    g1, dinv = _tc1b(degp, h1)
    t1 = sc_edge(g1, eir, zeros_h)
    g2 = _tc2(t1, g1, dinv, b1, W2)
    t2 = sc_edge(g2, eir, zeros_h)
    out = _tc3(t2, g2, dinv, b2, Wl, bl)
    return out[0]


# speedup vs baseline: 1.7124x; 1.0342x over previous
import functools

import jax
import jax.numpy as jnp
from jax import lax
from jax.experimental import pallas as pl
from jax.experimental.pallas import tpu as pltpu
from jax.experimental.pallas import tpu_sc as plsc

N = 10000
E = 320000
D = 128
DH = 64
DO = 64

NC = 2
NS = 16

CH = 125
EPT = E // NS
NCHUNK = EPT // CH
NBUF = 4

DNCHUNK = NCHUNK // NC

NP = 10240
SLAB = NP // NS

BLK = 1024
NBLK = NP // BLK


NR = NP // 128
RPT = NR // NS


def _sc_degree_body(ei_hbm, out_hbm, dst_v, cnt_v, rix_v, deg_sh):
    c = lax.axis_index("c")
    s = lax.axis_index("s")
    f32 = jnp.float32
    zeros16 = jnp.zeros((16,), f32)
    ones16 = jnp.ones((16,), f32)
    iota16 = lax.broadcasted_iota(jnp.int32, (16,), 0)

    def zrow(i, carry):
        for k in range(8):
            cnt_v[i, pl.ds(k * 16, 16)] = zeros16
        return carry

    lax.fori_loop(0, NR, zrow, 0)
    for k in range(NR // 16):
        rix_v[pl.ds(k * 16, 16)] = iota16 + (k * 16)
    pltpu.sync_copy(cnt_v.at[pl.ds(0, RPT)], deg_sh.at[pl.ds(s * RPT, RPT)])
    pltpu.sync_copy(ei_hbm.at[1, s, pl.ds(c * DNCHUNK, DNCHUNK)], dst_v)
    tailmask = iota16 >= 3

    def body(j, carry):
        for k in range(7):
            idx = dst_v[j, pl.ds(k * 16, 16)]
            plsc.addupdate_scatter(cnt_v, [idx >> 7, idx & 127], ones16)
        idx = dst_v[j, pl.ds(CH - 16, 16)]
        plsc.addupdate_scatter(cnt_v, [idx >> 7, idx & 127], ones16,
                               mask=tailmask)
        return carry

    lax.fori_loop(0, DNCHUNK, body, 0)
    plsc.subcore_barrier()
    pltpu.sync_copy(cnt_v, deg_sh.at[rix_v], add=True)
    plsc.subcore_barrier()
    pltpu.sync_copy(deg_sh.at[pl.ds(s * RPT, RPT)], out_hbm.at[c, pl.ds(s * RPT, RPT)])


def _sc_edge_body(g_hbm, ei_hbm, out_hbm, src_v, dst_v, rows, gsems, acc_sh):
    c = lax.axis_index("c")
    s = lax.axis_index("s")
    slab = s * SLAB
    zb = jnp.zeros((32,), jnp.bfloat16)

    def zbody(r, carry):
        rows[0][r, pl.ds(0, 32)] = zb
        rows[0][r, pl.ds(32, 32)] = zb
        return carry

    lax.fori_loop(0, CH, zbody, 0)
    for j in range(SLAB // CH):
        pltpu.sync_copy(rows[0], acc_sh.at[pl.ds(slab + j * CH, CH)])
    pltpu.sync_copy(rows[0].at[pl.ds(0, SLAB % CH)],
                    acc_sh.at[pl.ds(slab + (SLAB // CH) * CH, SLAB % CH)])
    pltpu.sync_copy(ei_hbm.at[0, s], src_v)
    pltpu.sync_copy(ei_hbm.at[1, s], dst_v)
    plsc.subcore_barrier()

    table = g_hbm.at[c]

    for b in range(NBUF):
        pltpu.async_copy(table.at[src_v.at[b]], rows[b], gsems[b])

    def body(grp, carry):
        for b in range(NBUF):
            ch = grp * NBUF + b
            pltpu.make_async_copy(table.at[src_v.at[ch]], rows[b],
                                  gsems[b]).wait()
            pltpu.sync_copy(rows[b], acc_sh.at[dst_v.at[ch]], add=True)

            @pl.when(ch + NBUF < NCHUNK)
            def _():
                pltpu.async_copy(table.at[src_v.at[ch + NBUF]], rows[b],
                                 gsems[b])
        return carry

    lax.fori_loop(0, NCHUNK // NBUF, body, 0)
    plsc.subcore_barrier()
    pltpu.sync_copy(acc_sh.at[pl.ds(slab, SLAB)],
                    out_hbm.at[pl.ds(slab, SLAB), pl.ds(c * DH, DH)])


@functools.lru_cache(maxsize=None)
def _sc_kernels():
    mesh = plsc.VectorSubcoreMesh(core_axis_name="c", subcore_axis_name="s",
                                  num_cores=NC, num_subcores=NS)
    params = pltpu.CompilerParams(use_tc_tiling_on_sc=False)
    sc_degree = pl.kernel(
        _sc_degree_body,
        out_type=jax.ShapeDtypeStruct((NC, NR, 128), jnp.float32),
        mesh=mesh,
        compiler_params=pltpu.CompilerParams(use_tc_tiling_on_sc=False,
                                             needs_layout_passes=False),
        scratch_types=[
            pltpu.VMEM((DNCHUNK, CH), jnp.int32),
            pltpu.VMEM((NR, 128), jnp.float32),
            pltpu.VMEM((NR,), jnp.int32),
            pltpu.VMEM_SHARED((NR, 128), jnp.float32),
        ],
    )
    sc_edge = pl.kernel(
        _sc_edge_body,
        out_type=jax.ShapeDtypeStruct((NP, D), jnp.bfloat16),
        mesh=mesh,
        compiler_params=params,
        scratch_types=[
            pltpu.VMEM((NCHUNK, CH), jnp.int32),
            pltpu.VMEM((NCHUNK, CH), jnp.int32),
            [pltpu.VMEM((CH, DH), jnp.bfloat16) for _ in range(NBUF)],
            [pltpu.SemaphoreType.DMA for _ in range(NBUF)],
            pltpu.VMEM_SHARED((NP, DH), jnp.bfloat16),
        ],
    )
    return sc_degree, sc_edge


def _tc1a_body(x_ref, w1_ref, h_ref):
    h_ref[...] = lax.dot_general(x_ref[...], w1_ref[...],
                                 (((1,), (1,)), ((), ())),
                                 preferred_element_type=jnp.float32)


_tc1a = pl.pallas_call(
    _tc1a_body,
    grid=(N // 1000,),
    in_specs=[
        pl.BlockSpec((1000, D), lambda i: (i, 0)),
        pl.BlockSpec((D, D), lambda i: (0, 0)),
    ],
    out_specs=pl.BlockSpec((1000, D), lambda i: (i, 0)),
    out_shape=jax.ShapeDtypeStruct((NP, D), jnp.float32),
)


def _tc1b_body(deg_ref, h_ref, g_ref, dinv_ref):
    deg = deg_ref[0] + deg_ref[1] + 1.0
    dv = lax.rsqrt(deg)
    g = (h_ref[...] * dv[:, None]).astype(jnp.bfloat16)
    g_ref[0] = g[:, :DH]
    g_ref[1] = g[:, DH:]
    dinv_ref[...] = dv


_tc1b = pl.pallas_call(
    _tc1b_body,
    grid=(NBLK,),
    in_specs=[
        pl.BlockSpec((NC, BLK), lambda i: (0, i)),
        pl.BlockSpec((BLK, D), lambda i: (i, 0)),
    ],
    out_specs=[
        pl.BlockSpec((NC, BLK, DH), lambda i: (0, i, 0)),
        pl.BlockSpec((BLK,), lambda i: (i,)),
    ],
    out_shape=[
        jax.ShapeDtypeStruct((NC, NP, DH), jnp.bfloat16),
        jax.ShapeDtypeStruct((NP,), jnp.float32),
    ],
)


def _tc2_body(t_ref, g_ref, dinv_ref, b1_ref, w2_ref, g2_ref):
    f32 = jnp.float32
    dv = dinv_ref[...]
    gfull = jnp.concatenate([g_ref[0], g_ref[1]], axis=1).astype(f32)
    t = t_ref[...].astype(f32) + gfull
    o1 = jnp.maximum(t * dv[:, None] + b1_ref[...][None, :], 0.0)
    h2 = lax.dot_general(o1, w2_ref[...], (((1,), (1,)), ((), ())),
                         preferred_element_type=jnp.float32)
    g2 = (h2 * dv[:, None]).astype(jnp.bfloat16)
    g2_ref[0] = g2[:, :DH]
    g2_ref[1] = g2[:, DH:]


_tc2 = pl.pallas_call(
    _tc2_body,
    grid=(NBLK,),
    in_specs=[
        pl.BlockSpec((BLK, D), lambda i: (i, 0)),
        pl.BlockSpec((NC, BLK, DH), lambda i: (0, i, 0)),
        pl.BlockSpec((BLK,), lambda i: (i,)),
        pl.BlockSpec((D,), lambda i: (0,)),
        pl.BlockSpec((D, D), lambda i: (0, 0)),
    ],
    out_specs=pl.BlockSpec((NC, BLK, DH), lambda i: (0, i, 0)),
    out_shape=jax.ShapeDtypeStruct((NC, NP, DH), jnp.bfloat16),
)


def _tc3_body(t_ref, g_ref, dinv_ref, b2_ref, wl_ref, bl_ref, out_ref):
    i = pl.program_id(0)
    f32 = jnp.float32
    dv = dinv_ref[...]
    gfull = jnp.concatenate([g_ref[0], g_ref[1]], axis=1).astype(f32)
    t = t_ref[...].astype(f32) + gfull
    o2 = jnp.maximum(t * dv[:, None] + b2_ref[...][None, :], 0.0)
    y = lax.dot_general(o2, wl_ref[...], (((1,), (1,)), ((), ())),
                        preferred_element_type=jnp.float32)
    y = jnp.maximum(y + bl_ref[...][None, :], 0.0)
    rid = i * BLK + lax.broadcasted_iota(jnp.int32, (BLK, 1), 0)
    y = jnp.where(rid < N, y, 0.0)
    ssum = jnp.sum(y, axis=0, keepdims=True)

    @pl.when(i == 0)
    def _():
        out_ref[...] = ssum

    @pl.when(i > 0)
    def _():
        out_ref[...] = out_ref[...] + ssum

    @pl.when(i == NBLK - 1)
    def _():
        out_ref[...] = jax.nn.sigmoid(out_ref[...])


_tc3 = pl.pallas_call(
    _tc3_body,
    grid=(NBLK,),
    in_specs=[
        pl.BlockSpec((BLK, D), lambda i: (i, 0)),
        pl.BlockSpec((NC, BLK, DH), lambda i: (0, i, 0)),
        pl.BlockSpec((BLK,), lambda i: (i,)),
        pl.BlockSpec((D,), lambda i: (0,)),
        pl.BlockSpec((DO, D), lambda i: (0, 0)),
        pl.BlockSpec((DO,), lambda i: (0,)),
    ],
    out_specs=pl.BlockSpec((1, DO), lambda i: (0, 0)),
    out_shape=jax.ShapeDtypeStruct((1, DO), jnp.float32),
)


def kernel(x, edge_index, batch, W1, b1, W2, b2, Wl, bl):
    eir = edge_index.reshape(2, NS, NCHUNK, CH)

    sc_degree, sc_edge = _sc_kernels()
    degp = sc_degree(eir).reshape(NC, NP)
    h1 = _tc1a(x, W1)
    g1, dinv = _tc1b(degp, h1)
    t1 = sc_edge(g1, eir)
    g2 = _tc2(t1, g1, dinv, b1, W2)
    t2 = sc_edge(g2, eir)
    out = _tc3(t2, g2, dinv, b2, Wl, bl)
    return out[0]


# speedup vs baseline: 1.8626x; 1.0877x over previous
import functools

import jax
import jax.numpy as jnp
from jax import lax
from jax.experimental import pallas as pl
from jax.experimental.pallas import tpu as pltpu
from jax.experimental.pallas import tpu_sc as plsc

N = 10000
E = 320000
D = 128
DH = 64
DO = 64

NC = 2
NS = 16

CH = 125
EPT = E // NS
NCHUNK = EPT // CH
NBUF = 8

DNCHUNK = NCHUNK // NC

NP = 10240
SLAB = NP // NS

BLK = 2048
NBLK = NP // BLK


NR = NP // 128
RPT = NR // NS


def _sc_degree_body(ei_hbm, out_hbm, dst_v, cnt_v, rix_v, deg_sh):
    c = lax.axis_index("c")
    s = lax.axis_index("s")
    f32 = jnp.float32
    zeros16 = jnp.zeros((16,), f32)
    ones16 = jnp.ones((16,), f32)
    iota16 = lax.broadcasted_iota(jnp.int32, (16,), 0)

    def zrow(i, carry):
        for k in range(8):
            cnt_v[i, pl.ds(k * 16, 16)] = zeros16
        return carry

    lax.fori_loop(0, NR, zrow, 0)
    for k in range(NR // 16):
        rix_v[pl.ds(k * 16, 16)] = iota16 + (k * 16)
    pltpu.sync_copy(cnt_v.at[pl.ds(0, RPT)], deg_sh.at[pl.ds(s * RPT, RPT)])
    pltpu.sync_copy(ei_hbm.at[1, s, pl.ds(c * DNCHUNK, DNCHUNK)], dst_v)
    tailmask = iota16 >= 3

    def body(j, carry):
        for k in range(7):
            idx = dst_v[j, pl.ds(k * 16, 16)]
            plsc.addupdate_scatter(cnt_v, [idx >> 7, idx & 127], ones16)
        idx = dst_v[j, pl.ds(CH - 16, 16)]
        plsc.addupdate_scatter(cnt_v, [idx >> 7, idx & 127], ones16,
                               mask=tailmask)
        return carry

    lax.fori_loop(0, DNCHUNK, body, 0)
    plsc.subcore_barrier()
    pltpu.sync_copy(cnt_v, deg_sh.at[rix_v], add=True)
    plsc.subcore_barrier()
    pltpu.sync_copy(deg_sh.at[pl.ds(s * RPT, RPT)], out_hbm.at[c, pl.ds(s * RPT, RPT)])


def _sc_edge_body(g_hbm, ei_hbm, out_hbm, src_v, dst_v, rows, gsems, acc_sh):
    c = lax.axis_index("c")
    s = lax.axis_index("s")
    slab = s * SLAB
    zb = jnp.zeros((32,), jnp.bfloat16)

    def zbody(r, carry):
        rows[0][r, pl.ds(0, 32)] = zb
        rows[0][r, pl.ds(32, 32)] = zb
        return carry

    lax.fori_loop(0, CH, zbody, 0)
    for j in range(SLAB // CH):
        pltpu.sync_copy(rows[0], acc_sh.at[pl.ds(slab + j * CH, CH)])
    pltpu.sync_copy(rows[0].at[pl.ds(0, SLAB % CH)],
                    acc_sh.at[pl.ds(slab + (SLAB // CH) * CH, SLAB % CH)])
    pltpu.sync_copy(ei_hbm.at[0, s], src_v)
    pltpu.sync_copy(ei_hbm.at[1, s], dst_v)
    plsc.subcore_barrier()

    table = g_hbm.at[c]

    for b in range(NBUF):
        pltpu.async_copy(table.at[src_v.at[b]], rows[b], gsems[b])

    def body(grp, carry):
        for b in range(NBUF):
            ch = grp * NBUF + b
            pltpu.make_async_copy(table.at[src_v.at[ch]], rows[b],
                                  gsems[b]).wait()
            pltpu.sync_copy(rows[b], acc_sh.at[dst_v.at[ch]], add=True)

            @pl.when(ch + NBUF < NCHUNK)
            def _():
                pltpu.async_copy(table.at[src_v.at[ch + NBUF]], rows[b],
                                 gsems[b])
        return carry

    lax.fori_loop(0, NCHUNK // NBUF, body, 0)
    plsc.subcore_barrier()
    pltpu.sync_copy(acc_sh.at[pl.ds(slab, SLAB)],
                    out_hbm.at[pl.ds(slab, SLAB), pl.ds(c * DH, DH)])


@functools.lru_cache(maxsize=None)
def _sc_kernels():
    mesh = plsc.VectorSubcoreMesh(core_axis_name="c", subcore_axis_name="s",
                                  num_cores=NC, num_subcores=NS)
    params = pltpu.CompilerParams(use_tc_tiling_on_sc=False)
    sc_degree = pl.kernel(
        _sc_degree_body,
        out_type=jax.ShapeDtypeStruct((NC, NR, 128), jnp.float32),
        mesh=mesh,
        compiler_params=pltpu.CompilerParams(use_tc_tiling_on_sc=False,
                                             needs_layout_passes=False),
        scratch_types=[
            pltpu.VMEM((DNCHUNK, CH), jnp.int32),
            pltpu.VMEM((NR, 128), jnp.float32),
            pltpu.VMEM((NR,), jnp.int32),
            pltpu.VMEM_SHARED((NR, 128), jnp.float32),
        ],
    )
    sc_edge = pl.kernel(
        _sc_edge_body,
        out_type=jax.ShapeDtypeStruct((NP, D), jnp.bfloat16),
        mesh=mesh,
        compiler_params=params,
        scratch_types=[
            pltpu.VMEM((NCHUNK, CH), jnp.int32),
            pltpu.VMEM((NCHUNK, CH), jnp.int32),
            [pltpu.VMEM((CH, DH), jnp.bfloat16) for _ in range(NBUF)],
            [pltpu.SemaphoreType.DMA for _ in range(NBUF)],
            pltpu.VMEM_SHARED((NP, DH), jnp.bfloat16),
        ],
    )
    return sc_degree, sc_edge


def _tc1a_body(x_ref, w1_ref, h_ref):
    h_ref[...] = lax.dot_general(x_ref[...], w1_ref[...],
                                 (((1,), (1,)), ((), ())),
                                 preferred_element_type=jnp.float32)


_tc1a = pl.pallas_call(
    _tc1a_body,
    grid=(N // 1000,),
    in_specs=[
        pl.BlockSpec((1000, D), lambda i: (i, 0)),
        pl.BlockSpec((D, D), lambda i: (0, 0)),
    ],
    out_specs=pl.BlockSpec((1000, D), lambda i: (i, 0)),
    out_shape=jax.ShapeDtypeStruct((NP, D), jnp.float32),
)


def _tc1b_body(deg_ref, h_ref, g_ref, dinv_ref):
    deg = deg_ref[0] + deg_ref[1] + 1.0
    dv = lax.rsqrt(deg)
    g = (h_ref[...] * dv[:, None]).astype(jnp.bfloat16)
    g_ref[0] = g[:, :DH]
    g_ref[1] = g[:, DH:]
    dinv_ref[...] = dv


_tc1b = pl.pallas_call(
    _tc1b_body,
    grid=(NBLK,),
    in_specs=[
        pl.BlockSpec((NC, BLK), lambda i: (0, i)),
        pl.BlockSpec((BLK, D), lambda i: (i, 0)),
    ],
    out_specs=[
        pl.BlockSpec((NC, BLK, DH), lambda i: (0, i, 0)),
        pl.BlockSpec((BLK,), lambda i: (i,)),
    ],
    out_shape=[
        jax.ShapeDtypeStruct((NC, NP, DH), jnp.bfloat16),
        jax.ShapeDtypeStruct((NP,), jnp.float32),
    ],
)


def _tc2_body(t_ref, g_ref, dinv_ref, b1_ref, w2_ref, g2_ref):
    f32 = jnp.float32
    dv = dinv_ref[...]
    gfull = jnp.concatenate([g_ref[0], g_ref[1]], axis=1).astype(f32)
    t = t_ref[...].astype(f32) + gfull
    o1 = jnp.maximum(t * dv[:, None] + b1_ref[...][None, :], 0.0)
    h2 = lax.dot_general(o1, w2_ref[...], (((1,), (1,)), ((), ())),
                         preferred_element_type=jnp.float32)
    g2 = (h2 * dv[:, None]).astype(jnp.bfloat16)
    g2_ref[0] = g2[:, :DH]
    g2_ref[1] = g2[:, DH:]


_tc2 = pl.pallas_call(
    _tc2_body,
    grid=(NBLK,),
    in_specs=[
        pl.BlockSpec((BLK, D), lambda i: (i, 0)),
        pl.BlockSpec((NC, BLK, DH), lambda i: (0, i, 0)),
        pl.BlockSpec((BLK,), lambda i: (i,)),
        pl.BlockSpec((D,), lambda i: (0,)),
        pl.BlockSpec((D, D), lambda i: (0, 0)),
    ],
    out_specs=pl.BlockSpec((NC, BLK, DH), lambda i: (0, i, 0)),
    out_shape=jax.ShapeDtypeStruct((NC, NP, DH), jnp.bfloat16),
)


def _tc3_body(t_ref, g_ref, dinv_ref, b2_ref, wl_ref, bl_ref, out_ref):
    i = pl.program_id(0)
    f32 = jnp.float32
    dv = dinv_ref[...]
    gfull = jnp.concatenate([g_ref[0], g_ref[1]], axis=1).astype(f32)
    t = t_ref[...].astype(f32) + gfull
    o2 = jnp.maximum(t * dv[:, None] + b2_ref[...][None, :], 0.0)
    y = lax.dot_general(o2, wl_ref[...], (((1,), (1,)), ((), ())),
                        preferred_element_type=jnp.float32)
    y = jnp.maximum(y + bl_ref[...][None, :], 0.0)
    rid = i * BLK + lax.broadcasted_iota(jnp.int32, (BLK, 1), 0)
    y = jnp.where(rid < N, y, 0.0)
    ssum = jnp.sum(y, axis=0, keepdims=True)

    @pl.when(i == 0)
    def _():
        out_ref[...] = ssum

    @pl.when(i > 0)
    def _():
        out_ref[...] = out_ref[...] + ssum

    @pl.when(i == NBLK - 1)
    def _():
        out_ref[...] = jax.nn.sigmoid(out_ref[...])


_tc3 = pl.pallas_call(
    _tc3_body,
    grid=(NBLK,),
    in_specs=[
        pl.BlockSpec((BLK, D), lambda i: (i, 0)),
        pl.BlockSpec((NC, BLK, DH), lambda i: (0, i, 0)),
        pl.BlockSpec((BLK,), lambda i: (i,)),
        pl.BlockSpec((D,), lambda i: (0,)),
        pl.BlockSpec((DO, D), lambda i: (0, 0)),
        pl.BlockSpec((DO,), lambda i: (0,)),
    ],
    out_specs=pl.BlockSpec((1, DO), lambda i: (0, 0)),
    out_shape=jax.ShapeDtypeStruct((1, DO), jnp.float32),
)


def kernel(x, edge_index, batch, W1, b1, W2, b2, Wl, bl):
    eir = edge_index.reshape(2, NS, NCHUNK, CH)

    sc_degree, sc_edge = _sc_kernels()
    degp = sc_degree(eir).reshape(NC, NP)
    h1 = _tc1a(x, W1)
    g1, dinv = _tc1b(degp, h1)
    t1 = sc_edge(g1, eir)
    g2 = _tc2(t1, g1, dinv, b1, W2)
    t2 = sc_edge(g2, eir)
    out = _tc3(t2, g2, dinv, b2, Wl, bl)
    return out[0]
